# Initial kernel scaffold; baseline (speedup 1.0000x reference)
import jax
import jax.numpy as jnp
from jax.experimental import pallas as pl


def kernel(x, W):
    raise NotImplementedError("write your pallas kernel here")


# speedup vs baseline: 1.3041x; 1.3041x over previous
import functools

import jax
import jax.numpy as jnp
from jax.experimental import pallas as pl
from jax.experimental.pallas import tpu as pltpu

_N_EMBD = 4096
_NUM_EXPERTS = 64
_TOP_K = 8
_BLOCK = 512


def _router_kernel(x_ref, w_ref, gates_ref, idx_ref, aux_ref,
                   cnt_ref, psum_ref, *, num_tokens, nblocks):
    i = pl.program_id(0)

    @pl.when(i == 0)
    def _init():
        cnt_ref[...] = jnp.zeros_like(cnt_ref)
        psum_ref[...] = jnp.zeros_like(psum_ref)

    x = x_ref[...]
    w = w_ref[...]
    logits = jax.lax.dot_general(
        x, w, (((1,), (1,)), ((), ())), preferred_element_type=jnp.float32)

    m = jnp.max(logits, axis=1, keepdims=True)
    e = jnp.exp(logits - m)
    denom = jnp.sum(e, axis=1, keepdims=True)
    probs = e / denom

    b = probs.shape[0]
    iota = jax.lax.broadcasted_iota(jnp.int32, (b, _NUM_EXPERTS), 1)
    work = probs
    sel_total = jnp.zeros((b, _NUM_EXPERTS), jnp.float32)
    gate_cols = []
    idx_cols = []
    for _ in range(_TOP_K):
        mv = jnp.max(work, axis=1, keepdims=True)
        is_max = work == mv
        idx_k = jnp.min(jnp.where(is_max, iota, _NUM_EXPERTS), axis=1,
                        keepdims=True)
        sel = iota == idx_k
        gate_cols.append(mv)
        idx_cols.append(idx_k)
        sel_total = sel_total + sel.astype(jnp.float32)
        work = jnp.where(sel, -jnp.inf, work)

    gates = jnp.concatenate(gate_cols, axis=1)
    gates = gates / (jnp.sum(gates, axis=1, keepdims=True) + 1e-9)
    gates_ref[...] = gates
    idx_ref[...] = jnp.concatenate(idx_cols, axis=1)

    cnt_ref[...] += jnp.sum(sel_total, axis=0, keepdims=True)
    psum_ref[...] += jnp.sum(probs, axis=0, keepdims=True)

    @pl.when(i == nblocks - 1)
    def _finalize():
        f = cnt_ref[...] / (num_tokens * _TOP_K + 1e-9)
        p = psum_ref[...] / num_tokens
        aux_ref[...] = _NUM_EXPERTS * jnp.sum(f * p, keepdims=True)


@jax.jit
def kernel(x, W):
    num_tokens = x.shape[0]
    nblocks = num_tokens // _BLOCK
    gates, idx, aux = pl.pallas_call(
        functools.partial(_router_kernel, num_tokens=num_tokens,
                          nblocks=nblocks),
        grid=(nblocks,),
        in_specs=[
            pl.BlockSpec((_BLOCK, _N_EMBD), lambda i: (i, 0)),
            pl.BlockSpec((_NUM_EXPERTS, _N_EMBD), lambda i: (0, 0)),
        ],
        out_specs=[
            pl.BlockSpec((_BLOCK, _TOP_K), lambda i: (i, 0)),
            pl.BlockSpec((_BLOCK, _TOP_K), lambda i: (i, 0)),
            pl.BlockSpec((1, 1), lambda i: (0, 0)),
        ],
        out_shape=[
            jax.ShapeDtypeStruct((num_tokens, _TOP_K), jnp.float32),
            jax.ShapeDtypeStruct((num_tokens, _TOP_K), jnp.int32),
            jax.ShapeDtypeStruct((1, 1), jnp.float32),
        ],
        scratch_shapes=[
            pltpu.VMEM((1, _NUM_EXPERTS), jnp.float32),
            pltpu.VMEM((1, _NUM_EXPERTS), jnp.float32),
        ],
    )(x, W)
    return gates, idx, aux[0, 0]


# speedup vs baseline: 1.6366x; 1.2549x over previous
import functools

import jax
import jax.numpy as jnp
from jax.experimental import pallas as pl
from jax.experimental.pallas import tpu as pltpu

_N_EMBD = 4096
_NUM_EXPERTS = 64
_TOP_K = 8
_BLOCK = 512


def _router_kernel(x_ref, w_ref, gates_ref, idx_ref, aux_ref,
                   cnt_ref, psum_ref, *, num_tokens, nblocks):
    i = pl.program_id(0)

    @pl.when(i == 0)
    def _init():
        cnt_ref[...] = jnp.zeros_like(cnt_ref)
        psum_ref[...] = jnp.zeros_like(psum_ref)

    x = x_ref[...]
    w = w_ref[...]
    logits = jax.lax.dot_general(
        w, x, (((1,), (1,)), ((), ())), preferred_element_type=jnp.float32)

    m = jnp.max(logits, axis=0, keepdims=True)
    e = jnp.exp(logits - m)
    denom = jnp.sum(e, axis=0, keepdims=True)
    probs = e / denom

    b = probs.shape[1]
    iota = jax.lax.broadcasted_iota(jnp.int32, (_NUM_EXPERTS, b), 0)
    bits = jax.lax.bitcast_convert_type(probs, jnp.int32)
    enc = jax.lax.bitcast_convert_type(
        (bits & ~0x3F) | (_NUM_EXPERTS - 1 - iota), jnp.float32)

    picks = []
    for _ in range(_TOP_K):
        mv = jnp.max(enc, axis=0, keepdims=True)
        picks.append(mv)
        enc = jnp.where(enc == mv, -1.0, enc)

    top = jnp.concatenate(picks, axis=0)
    top_bits = jax.lax.bitcast_convert_type(top, jnp.int32)
    idx_t = _NUM_EXPERTS - 1 - (top_bits & 0x3F)
    vals_t = jax.lax.bitcast_convert_type(top_bits & ~0x3F, jnp.float32)
    gates_t = vals_t / (jnp.sum(vals_t, axis=0, keepdims=True) + 1e-9)

    gates_ref[...] = gates_t.T
    idx_ref[...] = idx_t.T

    sel = (enc < 0).astype(jnp.float32)
    cnt_ref[...] += jnp.sum(sel, axis=1, keepdims=True)
    psum_ref[...] += jnp.sum(probs, axis=1, keepdims=True)

    @pl.when(i == nblocks - 1)
    def _finalize():
        f = cnt_ref[...] / (num_tokens * _TOP_K + 1e-9)
        p = psum_ref[...] / num_tokens
        aux_ref[...] = _NUM_EXPERTS * jnp.sum(f * p, keepdims=True)


@jax.jit
def kernel(x, W):
    num_tokens = x.shape[0]
    nblocks = num_tokens // _BLOCK
    gates, idx, aux = pl.pallas_call(
        functools.partial(_router_kernel, num_tokens=num_tokens,
                          nblocks=nblocks),
        grid=(nblocks,),
        in_specs=[
            pl.BlockSpec((_BLOCK, _N_EMBD), lambda i: (i, 0)),
            pl.BlockSpec((_NUM_EXPERTS, _N_EMBD), lambda i: (0, 0)),
        ],
        out_specs=[
            pl.BlockSpec((_BLOCK, _TOP_K), lambda i: (i, 0)),
            pl.BlockSpec((_BLOCK, _TOP_K), lambda i: (i, 0)),
            pl.BlockSpec((1, 1), lambda i: (0, 0)),
        ],
        out_shape=[
            jax.ShapeDtypeStruct((num_tokens, _TOP_K), jnp.float32),
            jax.ShapeDtypeStruct((num_tokens, _TOP_K), jnp.int32),
            jax.ShapeDtypeStruct((1, 1), jnp.float32),
        ],
        scratch_shapes=[
            pltpu.VMEM((_NUM_EXPERTS, 1), jnp.float32),
            pltpu.VMEM((_NUM_EXPERTS, 1), jnp.float32),
        ],
    )(x, W)
    return gates, idx, aux[0, 0]


# speedup vs baseline: 1.7467x; 1.0673x over previous
import functools

import jax
import jax.numpy as jnp
from jax.experimental import pallas as pl
from jax.experimental.pallas import tpu as pltpu

_N_EMBD = 4096
_NUM_EXPERTS = 64
_TOP_K = 8
_BLOCK = 1024
_KSPLIT = 2
_KCHUNK = _N_EMBD // _KSPLIT


def _router_kernel(x0_ref, x1_ref, w0_ref, w1_ref, gates_ref, idx_ref,
                   aux_ref, cnt_ref, psum_ref, *, num_tokens, nblocks):
    i = pl.program_id(0)

    @pl.when(i == 0)
    def _init():
        cnt_ref[...] = jnp.zeros_like(cnt_ref)
        psum_ref[...] = jnp.zeros_like(psum_ref)

    logits = jax.lax.dot_general(
        w0_ref[...], x0_ref[...], (((1,), (1,)), ((), ())),
        preferred_element_type=jnp.float32)
    logits = logits + jax.lax.dot_general(
        w1_ref[...], x1_ref[...], (((1,), (1,)), ((), ())),
        preferred_element_type=jnp.float32)

    m = jnp.max(logits, axis=0, keepdims=True)
    e = jnp.exp(logits - m)
    denom = jnp.sum(e, axis=0, keepdims=True)
    probs = e / denom

    b = probs.shape[1]
    iota = jax.lax.broadcasted_iota(jnp.int32, (_NUM_EXPERTS, b), 0)
    bits = jax.lax.bitcast_convert_type(probs, jnp.int32)
    enc = jax.lax.bitcast_convert_type(
        (bits & ~0x3F) | (_NUM_EXPERTS - 1 - iota), jnp.float32)

    picks = []
    for _ in range(_TOP_K):
        mv = jnp.max(enc, axis=0, keepdims=True)
        picks.append(mv)
        enc = jnp.where(enc == mv, -1.0, enc)

    top = jnp.concatenate(picks, axis=0)
    top_bits = jax.lax.bitcast_convert_type(top, jnp.int32)
    idx_t = _NUM_EXPERTS - 1 - (top_bits & 0x3F)
    vals_t = jax.lax.bitcast_convert_type(top_bits & ~0x3F, jnp.float32)
    gates_t = vals_t / (jnp.sum(vals_t, axis=0, keepdims=True) + 1e-9)

    gates_ref[...] = gates_t.T
    idx_ref[...] = idx_t.T

    sel = (enc < 0).astype(jnp.float32)
    cnt_ref[...] += jnp.sum(sel, axis=1, keepdims=True)
    psum_ref[...] += jnp.sum(probs, axis=1, keepdims=True)

    @pl.when(i == nblocks - 1)
    def _finalize():
        f = cnt_ref[...] / (num_tokens * _TOP_K + 1e-9)
        p = psum_ref[...] / num_tokens
        aux_ref[...] = _NUM_EXPERTS * jnp.sum(f * p, keepdims=True)


@jax.jit
def kernel(x, W):
    num_tokens = x.shape[0]
    nblocks = num_tokens // _BLOCK
    gates, idx, aux = pl.pallas_call(
        functools.partial(_router_kernel, num_tokens=num_tokens,
                          nblocks=nblocks),
        grid=(nblocks,),
        in_specs=[
            pl.BlockSpec((_BLOCK, _KCHUNK), lambda i: (i, 0)),
            pl.BlockSpec((_BLOCK, _KCHUNK), lambda i: (i, 1)),
            pl.BlockSpec((_NUM_EXPERTS, _KCHUNK), lambda i: (0, 0)),
            pl.BlockSpec((_NUM_EXPERTS, _KCHUNK), lambda i: (0, 1)),
        ],
        out_specs=[
            pl.BlockSpec((_BLOCK, _TOP_K), lambda i: (i, 0)),
            pl.BlockSpec((_BLOCK, _TOP_K), lambda i: (i, 0)),
            pl.BlockSpec((1, 1), lambda i: (0, 0)),
        ],
        out_shape=[
            jax.ShapeDtypeStruct((num_tokens, _TOP_K), jnp.float32),
            jax.ShapeDtypeStruct((num_tokens, _TOP_K), jnp.int32),
            jax.ShapeDtypeStruct((1, 1), jnp.float32),
        ],
        scratch_shapes=[
            pltpu.VMEM((_NUM_EXPERTS, 1), jnp.float32),
            pltpu.VMEM((_NUM_EXPERTS, 1), jnp.float32),
        ],
    )(x, x, W, W)
    return gates, idx, aux[0, 0]
